# Initial kernel scaffold; baseline (speedup 1.0000x reference)
import jax
import jax.numpy as jnp
from jax.experimental import pallas as pl


def kernel(x, w1, b1, w2, b2):
    raise NotImplementedError("write your pallas kernel here")


# speedup vs baseline: 1.8628x; 1.8628x over previous
import jax
import jax.numpy as jnp
from jax.experimental import pallas as pl
from jax.experimental.pallas import tpu as pltpu


def _fuse_kernel(w1_ref, b1_ref, w2_ref, b2_ref, wc_ref, bc_ref):
    wc_ref[...] = jnp.dot(w1_ref[...], w2_ref[...],
                          preferred_element_type=jnp.float32)
    bc = jnp.dot(b1_ref[...], w2_ref[...],
                 preferred_element_type=jnp.float32) + b2_ref[...]
    bc_ref[...] = jnp.broadcast_to(bc, bc_ref.shape)


def _fwd_kernel(x_ref, wc_ref, bc_ref, out_ref):
    y = jnp.dot(x_ref[...], wc_ref[...], preferred_element_type=jnp.float32)
    y = y + bc_ref[0:1, :]
    m = jnp.max(y, axis=1, keepdims=True)
    e = jnp.exp(y - m)
    out_ref[...] = (e / jnp.sum(e, axis=1, keepdims=True)).astype(out_ref.dtype)


def _fold_weights(w1, b1_2d, w2, b2_2d):
    D, H = w1.shape
    O = w2.shape[1]
    d_half = D // 2
    return pl.pallas_call(
        _fuse_kernel,
        out_shape=(
            jax.ShapeDtypeStruct((D, O), jnp.float32),
            jax.ShapeDtypeStruct((16, O), jnp.float32),
        ),
        grid=(2,),
        in_specs=[
            pl.BlockSpec((d_half, H), lambda i: (i, 0)),
            pl.BlockSpec((1, H), lambda i: (0, 0)),
            pl.BlockSpec((H, O), lambda i: (0, 0)),
            pl.BlockSpec((1, O), lambda i: (0, 0)),
        ],
        out_specs=(
            pl.BlockSpec((d_half, O), lambda i: (i, 0)),
            pl.BlockSpec((8, O), lambda i: (i, 0)),
        ),
        compiler_params=pltpu.CompilerParams(
            dimension_semantics=("parallel",),
            vmem_limit_bytes=60 << 20,
        ),
    )(w1, b1_2d, w2, b2_2d)


def _forward(x, wc, bc, tile_b):
    B, D = x.shape
    O = wc.shape[1]
    return pl.pallas_call(
        _fwd_kernel,
        out_shape=jax.ShapeDtypeStruct((B, O), jnp.float32),
        grid=(B // tile_b,),
        in_specs=[
            pl.BlockSpec((tile_b, D), lambda i: (i, 0)),
            pl.BlockSpec((D, O), lambda i: (0, 0)),
            pl.BlockSpec((8, O), lambda i: (0, 0)),
        ],
        out_specs=pl.BlockSpec((tile_b, O), lambda i: (i, 0)),
        compiler_params=pltpu.CompilerParams(
            dimension_semantics=("parallel",),
            vmem_limit_bytes=60 << 20,
        ),
    )(x, wc, bc)


def kernel(x, w1, b1, w2, b2):
    B, D = x.shape
    H = w1.shape[1]
    O = w2.shape[1]
    b1_2d = b1.reshape(1, H)
    b2_2d = b2.reshape(1, O)

    wc, bc = _fold_weights(w1, b1_2d, w2, b2_2d)

    tile_b = 1024
    while B % tile_b != 0 or (B // tile_b) % 2 != 0:
        tile_b //= 2
    return _forward(x, wc, bc, tile_b)


# speedup vs baseline: 1.9116x; 1.0262x over previous
import jax
import jax.numpy as jnp
from jax.experimental import pallas as pl
from jax.experimental.pallas import tpu as pltpu


def _fuse_kernel(w1_ref, b1_ref, w2_ref, b2_ref, wc_ref, bc_ref):
    wc_ref[...] = jnp.dot(w1_ref[...], w2_ref[...],
                          preferred_element_type=jnp.float32).astype(jnp.bfloat16)
    bc = jnp.dot(b1_ref[...], w2_ref[...],
                 preferred_element_type=jnp.float32) + b2_ref[...]
    bc_ref[...] = jnp.broadcast_to(bc, bc_ref.shape)


def _fwd_kernel(x_ref, wc_ref, bc_ref, out_ref):
    y = jnp.dot(x_ref[...].astype(jnp.bfloat16), wc_ref[...],
                preferred_element_type=jnp.float32)
    y = y + bc_ref[0:1, :]
    m = jnp.max(y, axis=1, keepdims=True)
    e = jnp.exp(y - m)
    out_ref[...] = (e / jnp.sum(e, axis=1, keepdims=True)).astype(out_ref.dtype)


def _fold_weights(w1, b1_2d, w2, b2_2d):
    D, H = w1.shape
    O = w2.shape[1]
    d_half = D // 2
    return pl.pallas_call(
        _fuse_kernel,
        out_shape=(
            jax.ShapeDtypeStruct((D, O), jnp.bfloat16),
            jax.ShapeDtypeStruct((16, O), jnp.float32),
        ),
        grid=(2,),
        in_specs=[
            pl.BlockSpec((d_half, H), lambda i: (i, 0)),
            pl.BlockSpec((1, H), lambda i: (0, 0)),
            pl.BlockSpec((H, O), lambda i: (0, 0)),
            pl.BlockSpec((1, O), lambda i: (0, 0)),
        ],
        out_specs=(
            pl.BlockSpec((d_half, O), lambda i: (i, 0)),
            pl.BlockSpec((8, O), lambda i: (i, 0)),
        ),
        compiler_params=pltpu.CompilerParams(
            dimension_semantics=("parallel",),
            vmem_limit_bytes=60 << 20,
        ),
    )(w1, b1_2d, w2, b2_2d)


def _forward(x, wc, bc, tile_b):
    B, D = x.shape
    O = wc.shape[1]
    return pl.pallas_call(
        _fwd_kernel,
        out_shape=jax.ShapeDtypeStruct((B, O), jnp.float32),
        grid=(B // tile_b,),
        in_specs=[
            pl.BlockSpec((tile_b, D), lambda i: (i, 0)),
            pl.BlockSpec((D, O), lambda i: (0, 0)),
            pl.BlockSpec((8, O), lambda i: (0, 0)),
        ],
        out_specs=pl.BlockSpec((tile_b, O), lambda i: (i, 0)),
        compiler_params=pltpu.CompilerParams(
            dimension_semantics=("parallel",),
            vmem_limit_bytes=60 << 20,
        ),
    )(x, wc, bc)


def kernel(x, w1, b1, w2, b2):
    B, D = x.shape
    H = w1.shape[1]
    O = w2.shape[1]
    b1_2d = b1.reshape(1, H)
    b2_2d = b2.reshape(1, O)

    wc, bc = _fold_weights(w1, b1_2d, w2, b2_2d)

    tile_b = 1024
    while B % tile_b != 0 or (B // tile_b) % 2 != 0:
        tile_b //= 2
    return _forward(x, wc, bc, tile_b)
